# baseline (speedup 1.0000x reference)
import functools

import jax
import jax.numpy as jnp
from jax import lax
from jax.experimental import pallas as pl
from jax.experimental.pallas import tpu as pltpu
from jax.experimental.pallas import tpu_sc as plsc

_N = 2048
_NW = 32


@functools.partial(jax.jit, static_argnames=('k', 'hp', 'chunk', 'n'))
def _sc_gather_max(y, idx_flat, *, k, hp, chunk=16, n=_N):
    nodes_w = n // _NW
    mesh = plsc.VectorSubcoreMesh(core_axis_name="c", subcore_axis_name="s")

    @functools.partial(
        pl.kernel, mesh=mesh,
        out_type=jax.ShapeDtypeStruct((n, hp), jnp.float32),
        scratch_types=[
            pltpu.VMEM((chunk * k,), jnp.int32),
            pltpu.VMEM((chunk * k, hp), jnp.float32),
            pltpu.VMEM((chunk, hp), jnp.float32),
            pltpu.SemaphoreType.DMA,
        ],
    )
    def body(y_hbm, idx_hbm, out_hbm, idx_v, rows_v, out_v, sem):
        wid = lax.axis_index("s") * 2 + lax.axis_index("c")
        base = wid * nodes_w

        def chunk_body(ci, carry):
            nb = base + ci * chunk
            pltpu.sync_copy(idx_hbm.at[pl.ds(nb * k, chunk * k)], idx_v)
            pltpu.async_copy(y_hbm.at[idx_v], rows_v, sem).wait()

            def node_body(n, carry2):
                def ch_body(c, carry3):
                    def red(j, acc):
                        return jnp.maximum(acc, rows_v[n * k + j, pl.ds(c * 16, 16)])
                    acc = rows_v[n * k, pl.ds(c * 16, 16)]
                    acc = lax.fori_loop(1, k, red, acc)
                    out_v[n, pl.ds(c * 16, 16)] = acc
                    return carry3
                return lax.fori_loop(0, hp // 16, ch_body, carry2)

            lax.fori_loop(0, chunk, node_body, carry)
            pltpu.sync_copy(out_v, out_hbm.at[pl.ds(nb, chunk)])
            return carry

        lax.fori_loop(0, nodes_w // chunk, chunk_body, 0)

    return body(y, idx_flat)


_ROWS = 256


def _lex_lt(ka, va, kb, vb):
    return (ka < kb) | ((ka == kb) & (va < vb))


def _pos_topk_body(pos_ref, post_ref, out_ref):
    a = pos_ref[...]
    bt = post_ref[...]
    ab = jnp.dot(a, bt, preferred_element_type=jnp.float32)
    na = jnp.sum(a * a, axis=1, keepdims=True)
    nb = jnp.sum(bt * bt, axis=0, keepdims=True)
    d = jnp.maximum(na - 2.0 * ab + nb, 0.0)
    j = lax.broadcasted_iota(jnp.int32, (_ROWS, _N), 1)
    keys, vals = d, j
    for p in range(1, 12):
        size = 1 << p
        up = (j & size) == 0

        def stage(s, carry, up=up, p=p):
            keys, vals = carry
            stride = jnp.int32(1) << (p - 1 - s)
            bit = (j & stride) != 0
            pk = jnp.where(bit, pltpu.roll(keys, stride, 1),
                           pltpu.roll(keys, _N - stride, 1))
            pv = jnp.where(bit, pltpu.roll(vals, stride, 1),
                           pltpu.roll(vals, _N - stride, 1))
            keep_small = up == (~bit)
            want_self = _lex_lt(keys, vals, pk, pv) == keep_small
            return (jnp.where(want_self, keys, pk),
                    jnp.where(want_self, vals, pv))

        keys, vals = lax.fori_loop(0, p, stage, (keys, vals))
    out_ref[...] = vals[:, :512]


def _pos_topk512(pos):
    return pl.pallas_call(
        _pos_topk_body,
        grid=(_N // _ROWS,),
        in_specs=[pl.BlockSpec((_ROWS, 3), lambda i: (i, 0)),
                  pl.BlockSpec((3, _N), lambda i: (0, 0))],
        out_specs=pl.BlockSpec((_ROWS, 512), lambda i: (i, 0)),
        out_shape=jax.ShapeDtypeStruct((_N, 512), jnp.int32),
    )(pos, pos.T)


def _feat_topk_body(x_ref, xt_ref, out_ref, *, kk):
    a = x_ref[...]
    bt = xt_ref[...]
    ab = jnp.dot(a, bt, preferred_element_type=jnp.float32)
    na = jnp.sum(a * a, axis=1, keepdims=True)
    nb = jnp.sum(bt * bt, axis=0, keepdims=True)
    d = jnp.maximum(na - 2.0 * ab + nb, 0.0)
    j = lax.broadcasted_iota(jnp.int32, (_ROWS, _N), 1)
    jc = lax.broadcasted_iota(jnp.int32, (_ROWS, kk), 1)
    out = jnp.zeros((_ROWS, kk), jnp.int32)

    def step(i, carry):
        d, out = carry
        m = jnp.min(d, axis=1, keepdims=True)
        cand = jnp.where(d == m, j, _N)
        sel = jnp.min(cand, axis=1, keepdims=True)
        out = jnp.where(jc == i, sel, out)
        d = jnp.where(j == sel, jnp.float32(jnp.inf), d)
        return d, out

    _, out = lax.fori_loop(0, kk, step, (d, out))
    out_ref[...] = out


def _feat_topk(x, kk):
    C = x.shape[-1]
    body = functools.partial(_feat_topk_body, kk=kk)
    return pl.pallas_call(
        body,
        grid=(_N // _ROWS,),
        in_specs=[pl.BlockSpec((_ROWS, C), lambda i: (i, 0)),
                  pl.BlockSpec((C, _N), lambda i: (0, 0))],
        out_specs=pl.BlockSpec((_ROWS, kk), lambda i: (i, 0)),
        out_shape=jax.ShapeDtypeStruct((_N, kk), jnp.int32),
    )(x, x.T)


def _dot(a, b):
    return jnp.dot(a, b, preferred_element_type=jnp.float32)


def _stn_body(x_ref, w1, b1, w2, b2, w3, b3, w4, b4, w5, b5, w6, b6, out_ref):
    x = x_ref[...]
    h = jax.nn.relu(_dot(x, w1[...]) + b1[...])
    h = jax.nn.relu(_dot(h, w2[...]) + b2[...])
    h = jax.nn.relu(_dot(h, w3[...]) + b3[...])
    g = jnp.max(h, axis=0, keepdims=True)
    g = jax.nn.relu(_dot(g, w4[...]) + b4[...])
    g = jax.nn.relu(_dot(g, w5[...]) + b5[...])
    t = _dot(g, w6[...]) + b6[...]
    xt = x
    for c in range(15):
        xt = xt + x[:, c:c + 1] * t[:, 15 * c:15 * (c + 1)]
    out_ref[...] = xt


def _stn15_pl(x, p):
    args = []
    for i in range(1, 7):
        args.append(p[f'stn_W{i}'])
        args.append(p[f'stn_b{i}'][None, :])
    return pl.pallas_call(
        _stn_body,
        out_shape=jax.ShapeDtypeStruct((_N, 15), jnp.float32),
    )(x[0], *args)[None]


def _pre_body(x_ref, w_ref, y_ref):
    y_ref[...] = _dot(x_ref[...], w_ref[...][0])


def _block_pre(xi, wys):
    C = xi.shape[-1]
    return pl.pallas_call(
        _pre_body,
        grid=(4,),
        in_specs=[pl.BlockSpec((_N, C), lambda s: (0, 0)),
                  pl.BlockSpec((1, C, 128), lambda s: (s, 0, 0))],
        out_specs=pl.BlockSpec((_N, 128), lambda s: (s, 0)),
        out_shape=jax.ShapeDtypeStruct((4 * _N, 128), jnp.float32),
    )(xi, wys)


def _post_body(g_ref, x_ref, wca, bca, wcb, bcb, wcc, bcc, wcd, bcd,
               wo1, wo2, bo, wo3, wo4, bo2, out_ref, *, dims):
    ha, hb, hc, hd, co1 = dims
    x = x_ref[...]
    g = g_ref[...]
    u = jax.nn.relu(g[0 * _N:1 * _N, :ha] + _dot(x, wca[...]) + bca[...])
    v = jax.nn.relu(g[1 * _N:2 * _N, :hb] + _dot(x, wcb[...]) + bcb[...])
    hbl = jax.nn.relu(_dot(u, wo1[...]) + _dot(v, wo2[...]) + bo[...])
    u2 = jax.nn.relu(g[2 * _N:3 * _N, :hc] + _dot(x, wcc[...]) + bcc[...])
    v2 = jax.nn.relu(g[3 * _N:4 * _N, :hd] + _dot(x, wcd[...]) + bcd[...])
    hfl = jax.nn.relu(_dot(u2, wo3[...]) + _dot(v2, wo4[...]) + bo2[...])
    out_ref[...] = jnp.concatenate([hbl, hfl], axis=1)


def _block_post(g4, xi, p, pre_b, pre_f):
    C = xi.shape[-1]
    args, dims = [], []
    for pre in (pre_b, pre_f):
        Wa, Wb = p[pre + '_Wa'], p[pre + '_Wb']
        dims += [Wa.shape[1], Wb.shape[1]]
        args += [Wa[:C] - Wa[C:], p[pre + '_ba'][None],
                 Wb[:C] - Wb[C:], p[pre + '_bb'][None]]
    wargs = []
    for pre in (pre_b, pre_f):
        Wo = p[pre + '_Wo']
        H = p[pre + '_Wa'].shape[1]
        wargs += [Wo[:H], Wo[H:], p[pre + '_bo'][None]]
    co1 = p[pre_b + '_Wo'].shape[1]
    co2 = p[pre_f + '_Wo'].shape[1]
    body = functools.partial(_post_body, dims=(*dims, co1))
    return pl.pallas_call(
        body,
        out_shape=jax.ShapeDtypeStruct((_N, co1 + co2), jnp.float32),
    )(g4, xi, *args, *wargs)


def _head_a_body(x1_ref, x2_ref, x3_ref, w1a, w1b, w1c, b1, w2, b2,
                 h_ref, gm_ref):
    h = jax.nn.relu(_dot(x1_ref[...], w1a[...]) + _dot(x2_ref[...], w1b[...])
                    + _dot(x3_ref[...], w1c[...]) + b1[...])
    h = jax.nn.relu(_dot(h, w2[...]) + b2[...])
    h_ref[...] = h
    gm_ref[...] = jnp.max(h, axis=0, keepdims=True)


def _head_b_body(h_ref, gm_ref, wfi, bfi,
                 r1w1, r1b1, r1w2, r1b2, r1wp, r1bp,
                 r2w1, r2b1, r2w2, r2b2, r2wp, r2bp, wout, bout, out_ref):
    w = jax.nn.sigmoid(_dot(gm_ref[...], wfi[...]) + bfi[...])
    h = h_ref[...] * w
    t = jax.nn.relu(_dot(h, r1w1[...]) + r1b1[...])
    t = _dot(t, r1w2[...]) + r1b2[...]
    h = jax.nn.relu(t + _dot(h, r1wp[...]) + r1bp[...])
    t = jax.nn.relu(_dot(h, r2w1[...]) + r2b1[...])
    t = _dot(t, r2w2[...]) + r2b2[...]
    h = jax.nn.relu(t + _dot(h, r2wp[...]) + r2bp[...])
    out_ref[...] = _dot(h, wout[...]) + bout[...]


def _head(x1, x2, x3, p):
    c1, c2 = x1.shape[-1], x2.shape[-1]
    W1 = p['mlp1_W']
    h, gm = pl.pallas_call(
        _head_a_body,
        out_shape=[jax.ShapeDtypeStruct((_N, 1024), jnp.float32),
                   jax.ShapeDtypeStruct((1, 1024), jnp.float32)],
    )(x1[0], x2[0], x3[0], W1[:c1], W1[c1:c1 + c2], W1[c1 + c2:],
      p['mlp1_b'][None], p['mlp2_W'], p['mlp2_b'][None])
    args = [p['fi_W'], p['fi_b'][None]]
    for r in ('r1', 'r2'):
        args += [p[r + '_W1'], p[r + '_b1'][None], p[r + '_W2'], p[r + '_b2'][None],
                 p[r + '_Wp'], p[r + '_bp'][None]]
    args += [p['out_W'], p['out_b'][None]]
    wspecs = [pl.BlockSpec(a.shape, lambda i: (0, 0)) for a in args]
    return pl.pallas_call(
        _head_b_body,
        grid=(_N // _ROWS,),
        in_specs=[pl.BlockSpec((_ROWS, 1024), lambda i: (i, 0)),
                  pl.BlockSpec((1, 1024), lambda i: (0, 0))] + wspecs,
        out_specs=pl.BlockSpec((_ROWS, 17), lambda i: (i, 0)),
        out_shape=jax.ShapeDtypeStruct((_N, 17), jnp.float32),
    )(h, gm, *args)[None]


def _pad32(idx):
    k = idx.shape[1]
    if k == 32:
        return idx
    return jnp.concatenate([idx] + [idx[:, -1:]] * (32 - k), axis=1)


_BLOCKS = (
    ('b1', 'f1', (0, 32), (0, 32), (0, 240, 20), (0, 12)),
    ('b2', 'f2', (16, 48), (14, 46), (6, 360, 36), (6, 18)),
    ('b3', 'f3', (16, 48), (14, 46), (6, 540, 54), (6, 18)),
)


def kernel(x, pos, params):
    p = params
    I = _pos_topk512(pos[0])
    xi = _stn15_pl(x, p)[0]

    xs = []
    for pre_b, pre_f, s_b, s_f, s_d, (lo, hi) in _BLOCKS:
        C = xi.shape[-1]
        wys = jnp.stack(
            [jnp.pad(p[pre + sfx][C:], ((0, 0), (0, 128 - p[pre + sfx].shape[1])))
             for pre, sfx in ((pre_b, '_Wa'), (pre_b, '_Wb'),
                              (pre_f, '_Wa'), (pre_f, '_Wb'))], axis=0)
        y4 = _block_pre(xi, wys)
        J = _feat_topk(xi, hi)[:, lo:hi]
        idx4 = jnp.concatenate(
            [_pad32(I[:, s_b[0]:s_b[1]]),
             _pad32(J) + _N,
             _pad32(I[:, s_f[0]:s_f[1]]) + 2 * _N,
             _pad32(I[:, s_d[0]:s_d[1]:s_d[2]]) + 3 * _N], axis=0)
        g4 = _sc_gather_max(y4, idx4.reshape(-1), k=32, hp=128, n=4 * _N)
        xi = _block_post(g4, xi, p, pre_b, pre_f)
        xs.append(xi)

    return _head(xs[0][None], xs[1][None], xs[2][None], p)
